# baseline (speedup 1.0000x reference)
import functools

import jax
import jax.numpy as jnp
import numpy as np
from jax import lax
from jax.experimental import pallas as pl
from jax.experimental.pallas import tpu as pltpu
from jax.experimental.pallas import tpu_sc as plsc

N_NODES = 100000
MAX_DEGREE = 64
BATCH = 1024
NS1 = 10
NS2 = 25

_info = plsc.get_sparse_core_info()
NC, NSUB, LANES = _info.num_cores, _info.num_subcores, _info.num_lanes
SEEDS_SC = BATCH // NC
F1_SC = SEEDS_SC * NS1
F2_SC = F1_SC * NS2
ND = 10
D_RANGE = F2_SC // ND
D_M = D_RANGE // NS2

PER1 = int(np.lcm(16, NS1))
PER2 = int(np.lcm(16, NS2))

_key = jax.random.key(42)
_key, _sub1 = jax.random.split(_key)
_PERM1 = np.asarray(jax.random.permutation(_sub1, MAX_DEGREE))
_key, _sub2 = jax.random.split(_key)
_PERM2 = np.asarray(jax.random.permutation(_sub2, MAX_DEGREE))
COLS1 = [int(c) for c in _PERM1[:NS1]]
COLS2 = [int(c) for c in _PERM2[:NS2]]

L1_OWNER = {15 - j: j for j in range(NS1)}
L2_OWNER = {}
for k in range(NS2):
    L2_OWNER.setdefault(k % NSUB, []).append(k)

OFF_F1 = 0
OFF_B1 = OFF_F1 + F1_SC
OFF_VD = OFF_B1 + NS1 * SEEDS_SC
OFF_D = OFF_B1
OFF_OUT = OFF_D + NS2 * D_M
D_HALF = D_RANGE // 2
ARENA = max(OFF_VD + F1_SC, OFF_OUT + D_HALF)

_M1 = np.arange(PER1)
_B1IDX = (OFF_B1 + (_M1 % NS1) * SEEDS_SC + _M1 // NS1).astype(np.int32)
_P2 = np.arange(PER2)
_DIDX = (OFF_D + (_P2 % NS2) * D_M + _P2 // NS2).astype(np.int32)


def _body(inputs_hbm, adjT_hbm, b1p_hbm, dp_hbm,
          out0_hbm, out1_hbm, out2_hbm, ex1_hbm, ex2_hbm,
          col_v, arena, b1p_v, dp_v, sem):
    sc = lax.axis_index("c")
    tid = lax.axis_index("s")

    pltpu.sync_copy(b1p_hbm, b1p_v)
    pltpu.sync_copy(dp_hbm, dp_v)
    sc_seed = pl.multiple_of(sc * SEEDS_SC, SEEDS_SC)

    for t, j in L1_OWNER.items():
        @pl.when(tid == t)
        def _(j=j):
            cpc = pltpu.async_copy(adjT_hbm.at[COLS1[j]], col_v, sem)
            pltpu.sync_copy(inputs_hbm.at[pl.ds(sc_seed, SEEDS_SC)],
                            arena.at[pl.ds(OFF_B1, SEEDS_SC)])
            cpc.wait()
            for ch in range(SEEDS_SC // LANES):
                ids = arena[pl.ds(OFF_B1 + ch * LANES, LANES)]
                arena[pl.ds(OFF_VD + ch * LANES, LANES)] = \
                    plsc.load_gather(col_v, [ids])
            pltpu.sync_copy(arena.at[pl.ds(OFF_VD, SEEDS_SC)],
                            ex1_hbm.at[sc].at[pl.ds(j * SEEDS_SC, SEEDS_SC)])

    @pl.when(tid == 15)
    def _():
        pltpu.sync_copy(arena.at[pl.ds(OFF_B1, SEEDS_SC)],
                        out0_hbm.at[pl.ds(sc_seed, SEEDS_SC)])

    plsc.subcore_barrier()

    pltpu.sync_copy(ex1_hbm.at[sc], arena.at[pl.ds(OFF_B1, NS1 * SEEDS_SC)])

    def bgrp(g, carry):
        for c in range(PER1 // LANES):
            bidx = b1p_v[pl.ds(c * LANES, LANES)] + g * (PER1 // NS1)
            arena[pl.ds(OFF_F1 + g * PER1 + c * LANES, LANES)] = \
                plsc.load_gather(arena, [bidx])
        return carry
    lax.fori_loop(0, F1_SC // PER1, bgrp, 0)

    @pl.when(tid == 0)
    def _():
        pltpu.sync_copy(arena.at[pl.ds(OFF_F1, F1_SC)],
                        out1_hbm.at[pl.ds(pl.multiple_of(sc * F1_SC, F1_SC),
                                          F1_SC)])

    for t, ks in L2_OWNER.items():
        @pl.when(tid == t)
        def _(ks=ks):
            for k in ks:
                pltpu.async_copy(adjT_hbm.at[COLS2[k]], col_v, sem).wait()

                def cgrp(g, carry):
                    for c in range(4):
                        off = g * 64 + c * LANES
                        f1 = arena[pl.ds(OFF_F1 + off, LANES)]
                        arena[pl.ds(OFF_VD + off, LANES)] = \
                            plsc.load_gather(col_v, [f1])
                    return carry
                lax.fori_loop(0, F1_SC // 64, cgrp, 0)
                pltpu.sync_copy(arena.at[pl.ds(OFF_VD, F1_SC)],
                                ex2_hbm.at[sc].at[pl.ds(k * F1_SC, F1_SC)])

    plsc.subcore_barrier()

    @pl.when(tid < ND)
    def _():
        for k in range(NS2):
            pltpu.sync_copy(
                ex2_hbm.at[sc].at[pl.ds(k * F1_SC + tid * D_M, D_M)],
                arena.at[pl.ds(OFF_D + k * D_M, D_M)])

        for h in range(2):
            def dgrp(g, carry, h=h):
                for c in range(PER2 // LANES):
                    didx = dp_v[pl.ds(c * LANES, LANES)] \
                        + (h * (D_HALF // PER2) + g) * (PER2 // NS2)
                    arena[pl.ds(OFF_OUT + g * PER2 + c * LANES, LANES)] = \
                        plsc.load_gather(arena, [didx])
                return carry
            lax.fori_loop(0, D_HALF // PER2, dgrp, 0)
            pltpu.sync_copy(
                arena.at[pl.ds(OFF_OUT, D_HALF)],
                out2_hbm.at[pl.ds(pl.multiple_of(
                    sc * F2_SC + tid * D_RANGE + h * D_HALF, D_HALF),
                    D_HALF)])


@functools.partial(
    pl.kernel,
    mesh=plsc.VectorSubcoreMesh(core_axis_name="c", subcore_axis_name="s"),
    out_type=(jax.ShapeDtypeStruct((BATCH,), jnp.int32),
              jax.ShapeDtypeStruct((BATCH * NS1,), jnp.int32),
              jax.ShapeDtypeStruct((BATCH * NS1 * NS2,), jnp.int32),
              jax.ShapeDtypeStruct((NC, NS1 * SEEDS_SC), jnp.int32),
              jax.ShapeDtypeStruct((NC, NS2 * F1_SC), jnp.int32)),
    scratch_types=[
        pltpu.VMEM((N_NODES,), jnp.int32),
        pltpu.VMEM((ARENA,), jnp.int32),
        pltpu.VMEM((PER1,), jnp.int32),
        pltpu.VMEM((PER2,), jnp.int32),
        pltpu.SemaphoreType.DMA,
    ],
    compiler_params=pltpu.CompilerParams(use_tc_tiling_on_sc=True,
                                         needs_layout_passes=False),
)
def _sample_kernel(inputs_hbm, adjT_hbm, b1p_hbm, dp_hbm,
                   out0_hbm, out1_hbm, out2_hbm, ex1_hbm, ex2_hbm, *scratch):
    _body(inputs_hbm, adjT_hbm, b1p_hbm, dp_hbm,
          out0_hbm, out1_hbm, out2_hbm, ex1_hbm, ex2_hbm, *scratch)


def kernel(inputs, adj_info):
    out0, out1, out2, _ex1, _ex2 = _sample_kernel(
        inputs, adj_info.T, jnp.asarray(_B1IDX), jnp.asarray(_DIDX))
    return (out0, out1, out2)


# speedup vs baseline: 1.1530x; 1.1530x over previous
import functools

import jax
import jax.numpy as jnp
import numpy as np
from jax import lax
from jax.experimental import pallas as pl
from jax.experimental.pallas import tpu as pltpu
from jax.experimental.pallas import tpu_sc as plsc

N_NODES = 100000
MAX_DEGREE = 64
BATCH = 1024
NS1 = 10
NS2 = 25

_info = plsc.get_sparse_core_info()
NC, NSUB, LANES = _info.num_cores, _info.num_subcores, _info.num_lanes
NW = NC * NSUB
IDS_W = BATCH // NW
F1_W = IDS_W * NS1
F2_W = F1_W * NS2
GCH = 80
G1 = F1_W // GCH
G2 = F2_W // GCH

PER1 = int(np.lcm(16, NS1))
PER2 = int(np.lcm(16, NS2))
ROWS_PER2 = PER2 // NS2

_key = jax.random.key(42)
_key, _sub1 = jax.random.split(_key)
_PERM1 = np.asarray(jax.random.permutation(_sub1, MAX_DEGREE))
_key, _sub2 = jax.random.split(_key)
_PERM2 = np.asarray(jax.random.permutation(_sub2, MAX_DEGREE))

_P1 = np.arange(PER1)
_P2 = np.arange(PER2)
_PM1 = (_P1 // NS1).astype(np.int32)
_PM2 = (_P2 // NS2).astype(np.int32)
_PC1F = (_PERM1[:NS1][_P1 % NS1] * N_NODES).astype(np.int32)
_PC2F = (_PERM2[:NS2][_P2 % NS2] * N_NODES).astype(np.int32)


def _body(inputs_hbm, adjf_hbm, pm1_hbm, pc1_hbm, pm2_hbm, pc2_hbm,
          out0_hbm, out1_hbm, out2_hbm,
          pm1_v, pc1_v, pm2_v, pc2_v, ids_v, idx1_v, f1_v, idx2_v, f2_v,
          sem, sem2):
    wid = lax.axis_index("s") * NC + lax.axis_index("c")

    pltpu.sync_copy(pm1_hbm, pm1_v)
    pltpu.sync_copy(pc1_hbm, pc1_v)
    pltpu.sync_copy(pm2_hbm, pm2_v)
    pltpu.sync_copy(pc2_hbm, pc2_v)
    base = pl.multiple_of(wid * IDS_W, IDS_W)
    pltpu.sync_copy(inputs_hbm.at[pl.ds(base, IDS_W)], ids_v)
    cp0 = pltpu.async_copy(ids_v, out0_hbm.at[pl.ds(base, IDS_W)], sem2)

    for i in range(F1_W // LANES):
        m = pm1_v[pl.ds((i % (PER1 // LANES)) * LANES, LANES)] \
            + (i // (PER1 // LANES)) * (PER1 // NS1)
        seed = plsc.load_gather(ids_v, [m])
        cf = pc1_v[pl.ds((i % (PER1 // LANES)) * LANES, LANES)]
        idx1_v[i // (GCH // LANES),
               pl.ds((i % (GCH // LANES)) * LANES, LANES)] = seed + cf

    cps1 = [
        pltpu.async_copy(adjf_hbm.at[idx1_v.at[j]],
                         f1_v.at[pl.ds(j * GCH, GCH)], sem)
        for j in range(G1)
    ]
    for cp in cps1:
        cp.wait()

    cpo1 = pltpu.async_copy(
        f1_v, out1_hbm.at[pl.ds(pl.multiple_of(wid * F1_W, F1_W), F1_W)], sem2)

    def grp(g, carry):
        for c in range(PER2 // LANES):
            m = pm2_v[pl.ds(c * LANES, LANES)] + g * ROWS_PER2
            node = plsc.load_gather(f1_v, [m])
            cf = pc2_v[pl.ds(c * LANES, LANES)]
            idx2_v[g * (PER2 // GCH) + c // (GCH // LANES),
                   pl.ds((c % (GCH // LANES)) * LANES, LANES)] = node + cf
        for j in range(PER2 // GCH):
            row = g * (PER2 // GCH) + j
            pltpu.async_copy(adjf_hbm.at[idx2_v.at[row]],
                             f2_v.at[pl.ds(g * PER2 + j * GCH, GCH)], sem)
        return carry

    lax.fori_loop(0, F2_W // PER2, grp, 0)

    pltpu.make_async_copy(out2_hbm.at[pl.ds(0, F2_W)], f2_v, sem).wait()

    pltpu.sync_copy(f2_v,
                    out2_hbm.at[pl.ds(pl.multiple_of(wid * F2_W, F2_W), F2_W)])
    cp0.wait()
    cpo1.wait()


@functools.partial(
    pl.kernel,
    mesh=plsc.VectorSubcoreMesh(core_axis_name="c", subcore_axis_name="s"),
    out_type=(jax.ShapeDtypeStruct((BATCH,), jnp.int32),
              jax.ShapeDtypeStruct((BATCH * NS1,), jnp.int32),
              jax.ShapeDtypeStruct((BATCH * NS1 * NS2,), jnp.int32)),
    scratch_types=[
        pltpu.VMEM((PER1,), jnp.int32),
        pltpu.VMEM((PER1,), jnp.int32),
        pltpu.VMEM((PER2,), jnp.int32),
        pltpu.VMEM((PER2,), jnp.int32),
        pltpu.VMEM((IDS_W,), jnp.int32),
        pltpu.VMEM((G1, GCH), jnp.int32),
        pltpu.VMEM((F1_W,), jnp.int32),
        pltpu.VMEM((G2, GCH), jnp.int32),
        pltpu.VMEM((F2_W,), jnp.int32),
        pltpu.SemaphoreType.DMA,
        pltpu.SemaphoreType.DMA,
    ],
    compiler_params=pltpu.CompilerParams(use_tc_tiling_on_sc=False,
                                         needs_layout_passes=False),
)
def _sample_kernel(inputs_hbm, adjf_hbm, pm1_hbm, pc1_hbm, pm2_hbm, pc2_hbm,
                   out0_hbm, out1_hbm, out2_hbm, *scratch):
    _body(inputs_hbm, adjf_hbm, pm1_hbm, pc1_hbm, pm2_hbm, pc2_hbm,
          out0_hbm, out1_hbm, out2_hbm, *scratch)


def kernel(inputs, adj_info):
    adjf = jnp.reshape(adj_info.T, (N_NODES * MAX_DEGREE,))
    out0, out1, out2 = _sample_kernel(inputs, adjf,
                                      jnp.asarray(_PM1), jnp.asarray(_PC1F),
                                      jnp.asarray(_PM2), jnp.asarray(_PC2F))
    return (out0, out1, out2)


# speedup vs baseline: 1.1645x; 1.0100x over previous
import functools

import jax
import jax.numpy as jnp
import numpy as np
from jax import lax
from jax.experimental import pallas as pl
from jax.experimental.pallas import tpu as pltpu
from jax.experimental.pallas import tpu_sc as plsc

N_NODES = 100000
MAX_DEGREE = 64
BATCH = 1024
NS1 = 10
NS2 = 25

_info = plsc.get_sparse_core_info()
NC, NSUB, LANES = _info.num_cores, _info.num_subcores, _info.num_lanes
NW = NC * NSUB
SEEDS_SC = BATCH // NC
F1_SC = SEEDS_SC * NS1
F1_W = BATCH * NS1 // NW
F2_W = F1_W * NS2
GCH = 80
G1 = F1_W // GCH

PER1 = int(np.lcm(16, NS1))
PER2 = int(np.lcm(16, NS2))
ROWS_PER2 = PER2 // NS2

_key = jax.random.key(42)
_key, _sub1 = jax.random.split(_key)
_PERM1 = np.asarray(jax.random.permutation(_sub1, MAX_DEGREE))
_key, _sub2 = jax.random.split(_key)
_PERM2 = np.asarray(jax.random.permutation(_sub2, MAX_DEGREE))
COLS1 = [int(c) for c in _PERM1[:NS1]]

L1_OWNER = {15 - j: j for j in range(NS1)}

OFF_F1 = 0
OFF_B1 = OFF_F1 + F1_SC
OFF_VD = OFF_B1 + NS1 * SEEDS_SC
ARENA1 = OFF_VD + F1_SC

_M1 = np.arange(PER1)
_B1IDX = (OFF_B1 + (_M1 % NS1) * SEEDS_SC + _M1 // NS1).astype(np.int32)

_P2 = np.arange(PER2)
_PM2 = (_P2 // NS2).astype(np.int32)
_PC2F = (_PERM2[:NS2][_P2 % NS2] * N_NODES).astype(np.int32)


@functools.partial(
    pl.kernel,
    mesh=plsc.VectorSubcoreMesh(core_axis_name="c", subcore_axis_name="s"),
    out_type=(jax.ShapeDtypeStruct((BATCH,), jnp.int32),
              jax.ShapeDtypeStruct((BATCH * NS1,), jnp.int32),
              jax.ShapeDtypeStruct((NC, NS1 * SEEDS_SC), jnp.int32)),
    scratch_types=[
        pltpu.VMEM((N_NODES,), jnp.int32),
        pltpu.VMEM((ARENA1,), jnp.int32),
        pltpu.VMEM((PER1,), jnp.int32),
        pltpu.SemaphoreType.DMA,
    ],
    compiler_params=pltpu.CompilerParams(use_tc_tiling_on_sc=True,
                                         needs_layout_passes=False),
)
def _layer1_kernel(inputs_hbm, adjT_hbm, b1p_hbm,
                   out0_hbm, out1_hbm, ex1_hbm,
                   col_v, arena, b1p_v, sem):
    sc = lax.axis_index("c")
    tid = lax.axis_index("s")
    pltpu.sync_copy(b1p_hbm, b1p_v)
    sc_seed = pl.multiple_of(sc * SEEDS_SC, SEEDS_SC)

    for t, j in L1_OWNER.items():
        @pl.when(tid == t)
        def _(j=j):
            cpc = pltpu.async_copy(adjT_hbm.at[COLS1[j]], col_v, sem)
            pltpu.sync_copy(inputs_hbm.at[pl.ds(sc_seed, SEEDS_SC)],
                            arena.at[pl.ds(OFF_B1, SEEDS_SC)])
            cpc.wait()
            for ch in range(SEEDS_SC // LANES):
                ids = arena[pl.ds(OFF_B1 + ch * LANES, LANES)]
                arena[pl.ds(OFF_VD + ch * LANES, LANES)] = \
                    plsc.load_gather(col_v, [ids])
            pltpu.sync_copy(arena.at[pl.ds(OFF_VD, SEEDS_SC)],
                            ex1_hbm.at[sc].at[pl.ds(j * SEEDS_SC, SEEDS_SC)])

    @pl.when(tid == 15)
    def _():
        pltpu.sync_copy(arena.at[pl.ds(OFF_B1, SEEDS_SC)],
                        out0_hbm.at[pl.ds(sc_seed, SEEDS_SC)])

    plsc.subcore_barrier()

    @pl.when(tid == 0)
    def _():
        pltpu.sync_copy(ex1_hbm.at[sc],
                        arena.at[pl.ds(OFF_B1, NS1 * SEEDS_SC)])

        def bgrp(g, carry):
            for c in range(PER1 // LANES):
                bidx = b1p_v[pl.ds(c * LANES, LANES)] + g * (PER1 // NS1)
                arena[pl.ds(OFF_F1 + g * PER1 + c * LANES, LANES)] = \
                    plsc.load_gather(arena, [bidx])
            return carry
        lax.fori_loop(0, F1_SC // PER1, bgrp, 0)
        pltpu.sync_copy(arena.at[pl.ds(OFF_F1, F1_SC)],
                        out1_hbm.at[pl.ds(pl.multiple_of(sc * F1_SC, F1_SC),
                                          F1_SC)])


@functools.partial(
    pl.kernel,
    mesh=plsc.VectorSubcoreMesh(core_axis_name="c", subcore_axis_name="s"),
    out_type=jax.ShapeDtypeStruct((BATCH * NS1 * NS2,), jnp.int32),
    scratch_types=[
        pltpu.VMEM((PER2,), jnp.int32),
        pltpu.VMEM((PER2,), jnp.int32),
        pltpu.VMEM((F1_W,), jnp.int32),
        pltpu.VMEM((F2_W // GCH, GCH), jnp.int32),
        pltpu.VMEM((F2_W,), jnp.int32),
        pltpu.SemaphoreType.DMA,
    ],
    compiler_params=pltpu.CompilerParams(use_tc_tiling_on_sc=False,
                                         needs_layout_passes=False),
)
def _layer2_kernel(f1_hbm, adjf_hbm, pm2_hbm, pc2_hbm, out2_hbm,
                   pm2_v, pc2_v, f1_v, idx2_v, f2_v, sem):
    wid = lax.axis_index("s") * NC + lax.axis_index("c")
    pltpu.sync_copy(pm2_hbm, pm2_v)
    pltpu.sync_copy(pc2_hbm, pc2_v)
    pltpu.sync_copy(f1_hbm.at[pl.ds(pl.multiple_of(wid * F1_W, F1_W), F1_W)],
                    f1_v)

    def grp(g, carry):
        for c in range(PER2 // LANES):
            m = pm2_v[pl.ds(c * LANES, LANES)] + g * ROWS_PER2
            node = plsc.load_gather(f1_v, [m])
            cf = pc2_v[pl.ds(c * LANES, LANES)]
            idx2_v[g * (PER2 // GCH) + c // (GCH // LANES),
                   pl.ds((c % (GCH // LANES)) * LANES, LANES)] = node + cf
        for j in range(PER2 // GCH):
            row = g * (PER2 // GCH) + j
            pltpu.async_copy(adjf_hbm.at[idx2_v.at[row]],
                             f2_v.at[pl.ds(g * PER2 + j * GCH, GCH)], sem)
        return carry

    lax.fori_loop(0, F2_W // PER2, grp, 0)
    pltpu.make_async_copy(out2_hbm.at[pl.ds(0, F2_W)], f2_v, sem).wait()
    pltpu.sync_copy(f2_v,
                    out2_hbm.at[pl.ds(pl.multiple_of(wid * F2_W, F2_W), F2_W)])


def kernel(inputs, adj_info):
    adjT = adj_info.T
    out0, out1, _ex1 = _layer1_kernel(inputs, adjT, jnp.asarray(_B1IDX))
    adjf = jnp.reshape(adjT, (N_NODES * MAX_DEGREE,))
    out2 = _layer2_kernel(out1, adjf, jnp.asarray(_PM2), jnp.asarray(_PC2F))
    return (out0, out1, out2)
